# Initial kernel scaffold; baseline (speedup 1.0000x reference)
import jax
import jax.numpy as jnp
from jax.experimental import pallas as pl


def kernel(x_LE, labels, w1, w2, miu, tao, weight):
    raise NotImplementedError("write your pallas kernel here")


# speedup vs baseline: 1.9316x; 1.9316x over previous
import functools

import jax
import jax.numpy as jnp
from jax import lax
from jax.experimental import pallas as pl
from jax.experimental.pallas import tpu as pltpu
from jax.experimental.pallas import tpu_sc as plsc

_C = 16
_D = 8
_B = 32
_IN = 64
_H = 16
_W = 16
_OUT = 64
_Q = _OUT * _H * _W
_P4 = 4 * _Q
_EPS = 1e-6
_NW = 32
_CHUNK = _P4 // _NW


def _ls(x):
    return jnp.minimum(x, 0.0) - jnp.log1p(jnp.exp(-jnp.abs(x)))


def _sc_hist(x2d, labi):
    mesh = plsc.VectorSubcoreMesh(core_axis_name="c", subcore_axis_name="s")

    @functools.partial(
        pl.kernel,
        out_type=[
            jax.ShapeDtypeStruct((4, _C, _Q), jnp.float32),
            jax.ShapeDtypeStruct((_C,), jnp.float32),
        ],
        mesh=mesh,
        scratch_types=[
            pltpu.VMEM((_B,), jnp.int32),
            pltpu.VMEM((_B, _CHUNK), jnp.float32),
            pltpu.VMEM((_C, _CHUNK), jnp.float32),
            pltpu.VMEM((_C,), jnp.float32),
        ],
    )
    def hist(x_hbm, lab_hbm, sums_hbm, cnt_hbm, labs_v, xbuf, acc, cntv):
        wid = lax.axis_index("c") * 16 + lax.axis_index("s")
        start = wid * _CHUNK
        ch = start // _Q
        qoff = start % _Q
        pltpu.sync_copy(lab_hbm, labs_v)
        pltpu.sync_copy(x_hbm.at[:, pl.ds(start, _CHUNK)], xbuf)
        iot = lax.broadcasted_iota(jnp.int32, (16,), 0)
        zero16 = jnp.zeros((16,), jnp.float32)

        def zbody(i, carry):
            for r in range(_C):
                acc[r, pl.ds(i * 16, 16)] = zero16
            return carry

        lax.fori_loop(0, _CHUNK // 16, zbody, 0)

        lab_lo = labs_v[pl.ds(0, 16)]
        lab_hi = labs_v[pl.ds(16, 16)]
        labs = [lab_lo[i] for i in range(16)] + [lab_hi[i] for i in range(16)]

        def jbody(j, carry):
            sl = pl.ds(j * 16, 16)
            for b in range(_B):
                acc[labs[b], sl] = acc[labs[b], sl] + xbuf[b, sl]
            return carry

        lax.fori_loop(0, _CHUNK // 16, jbody, 0)
        pltpu.sync_copy(acc, sums_hbm.at[ch, :, pl.ds(qoff, _CHUNK)])

        @pl.when(wid == 0)
        def _():
            cnt = jnp.zeros((16,), jnp.float32)
            for b in range(_B):
                cnt = cnt + jnp.where(iot == labs[b], 1.0, 0.0)
            cntv[...] = cnt
            pltpu.sync_copy(cntv, cnt_hbm)

    return hist(x2d, labi)


def _tc_body(x_ref, s4_ref, cnt_ref, miu_ref, par_ref, out_ref, loss_ref,
             mth_ref, lm_ref, xy0_ref, xy1_ref):
    b = pl.program_id(0)

    @pl.when(b == 0)
    def _stats():
        xw = cnt_ref[...] + _EPS
        mag = (s4_ref[1] + _EPS) / xw
        rot = (s4_ref[0] + _EPS) / xw
        xy0_ref[...] = (s4_ref[2] + _EPS) / xw
        xy1_ref[...] = (s4_ref[3] + _EPS) / xw
        lmag = _ls(mag + _EPS)
        a_rot = _ls(rot)
        a_mag = _ls(mag)
        mm = jnp.zeros((_C, _Q), jnp.float32)
        mth0 = jnp.zeros((1, _Q), jnp.float32)
        lossm = jnp.zeros((_D, 128), jnp.float32)
        for d in range(_D):
            al = par_ref[d]
            be = par_ref[8 + d]
            md = miu_ref[d]
            m0 = md[0:1, :]
            m1 = md[1:2, :]
            mth0 = mth0 + be * m0
            mm = mm + jnp.exp(be * _ls(m1 + _EPS)) * jnp.exp(al * lmag)
            nrm = (jnp.sum((a_rot - _ls(m0)) ** 2, axis=1, keepdims=True)
                   + jnp.sum((a_mag - _ls(m1)) ** 2, axis=1, keepdims=True))
            t3 = par_ref[16 + d] / xw
            lossd = par_ref[24 + d] * jnp.mean(nrm + t3)
            rowm = lax.broadcasted_iota(jnp.int32, (_D, 128), 0) == d
            lossm = jnp.where(rowm, lossd, lossm)
        mth_ref[...] = par_ref[32] * mag + mth0
        lm_ref[...] = jnp.log(mm + _EPS)
        loss_ref[...] = lossm

    x = x_ref[0]
    lx1 = jnp.log(x[1:2, :])
    dr = jnp.abs(x[0:1, :] - mth_ref[...])
    da = jnp.abs(lx1 - lm_ref[...])
    dxy = (x[2:3, :] - xy0_ref[...]) ** 2 + (x[3:4, :] - xy1_ref[...]) ** 2
    dist = par_ref[33] * dr + par_ref[34] * da + par_ref[35] * dxy
    out_ref[0] = jnp.min(dist, axis=0, keepdims=True)


def _tc_main(x4, sums4, cnt, miu2, params):
    return pl.pallas_call(
        _tc_body,
        grid=(_B,),
        in_specs=[
            pl.BlockSpec((1, 4, _Q), lambda b: (b, 0, 0)),
            pl.BlockSpec((4, _C, _Q), lambda b: (0, 0, 0)),
            pl.BlockSpec((_C, 1), lambda b: (0, 0)),
            pl.BlockSpec((_D, 2, _Q), lambda b: (0, 0, 0)),
            pl.BlockSpec(memory_space=pltpu.SMEM),
        ],
        out_specs=[
            pl.BlockSpec((1, 1, _Q), lambda b: (b, 0, 0)),
            pl.BlockSpec((_D, 128), lambda b: (0, 0)),
        ],
        out_shape=[
            jax.ShapeDtypeStruct((_B, 1, _Q), jnp.float32),
            jax.ShapeDtypeStruct((_D, 128), jnp.float32),
        ],
        scratch_shapes=[pltpu.VMEM((_C, _Q), jnp.float32)] * 4,
    )(x4, sums4, cnt, miu2, params)


def kernel(x_LE, labels, w1, w2, miu, tao, weight):
    x2d = x_LE.reshape(_B, _P4)
    labi = labels.astype(jnp.int32)
    sums4, cnt = _sc_hist(x2d, labi)

    w1n = w1 ** 2 / jnp.sum(w1 ** 2)
    t2 = tao ** 2
    ft = t2 / (1.0 + t2)
    fs = 1.0 / (1.0 + t2)
    alpha = w1n * ft
    beta = w1n * fs
    t3coef = (2.0 * _Q) * (t2 ** 2 - 1.0)
    term1 = 1.0 / (1.0 + t2) ** 2
    a_sum = jnp.sum(alpha)[None]
    wsq = weight ** 2
    params = jnp.concatenate(
        [alpha, beta, t3coef, term1, a_sum, wsq, jnp.zeros((4,), jnp.float32)]
    ).astype(jnp.float32)

    out2, loss_pad = _tc_main(
        x_LE.reshape(_B, 4, _Q), sums4, cnt.reshape(_C, 1),
        miu.reshape(_D, 2, _Q), params)
    out = out2.reshape(_B, _IN, _H, _W)
    loss = loss_pad[:, 0]
    return out, loss
